# baseline (speedup 1.0000x reference)
import functools

import jax
import jax.numpy as jnp
from jax import lax
from jax.experimental import pallas as pl
from jax.experimental.pallas import tpu as pltpu
from jax.experimental.pallas import tpu_sc as plsc

B = 16384
D = 64
V = 1000000
NC = 2
NS = 16
NW = NC * NS
BPW = B // NW
CHUNK = 128
NCH = BPW // CHUNK
L = 16
TBLK = 512


def _transpose_body(inT_ref, out_ref):
    eye = (lax.broadcasted_iota(jnp.int32, (D, D), 0)
           == lax.broadcasted_iota(jnp.int32, (D, D), 1)).astype(jnp.float32)
    out_ref[...] = lax.dot_general(
        inT_ref[...], eye, (((0,), (0,)), ((), ())),
        precision=lax.Precision.HIGHEST,
        preferred_element_type=jnp.float32)


_transpose_tc = pl.pallas_call(
    _transpose_body,
    out_shape=jax.ShapeDtypeStruct((V, D), jnp.float32),
    grid=(V // TBLK,),
    in_specs=[pl.BlockSpec((D, TBLK), lambda i: (0, i))],
    out_specs=pl.BlockSpec((TBLK, D), lambda i: (i, 0)),
)

_mesh = plsc.VectorSubcoreMesh(core_axis_name="c", subcore_axis_name="s")


@functools.partial(
    pl.kernel,
    out_type=jax.ShapeDtypeStruct((B,), jnp.float32),
    mesh=_mesh,
    compiler_params=pltpu.CompilerParams(
        needs_layout_passes=False, use_tc_tiling_on_sc=False),
    scratch_types=[
        pltpu.VMEM((BPW,), jnp.int32),
        pltpu.VMEM((BPW,), jnp.int32),
        pltpu.VMEM((CHUNK, D), jnp.float32),
        pltpu.VMEM((CHUNK, D), jnp.float32),
        pltpu.VMEM((BPW,), jnp.float32),
        pltpu.VMEM((BPW,), jnp.float32),
        pltpu.VMEM((BPW,), jnp.float32),
        pltpu.SemaphoreType.DMA,
    ],
)
def _mf_kernel(user_hbm, item_hbm, uf_hbm, if_hbm, ub_hbm, ib_hbm, out_hbm,
               uidx_v, iidx_v, urows_v, irows_v, ub_v, ib_v, out_v, sem):
    wid = lax.axis_index("s") * NC + lax.axis_index("c")
    base = wid * BPW

    pltpu.sync_copy(user_hbm.at[pl.ds(base, BPW)], uidx_v)
    pltpu.sync_copy(item_hbm.at[pl.ds(base, BPW)], iidx_v)

    for c in range(NCH):
        s = pl.ds(c * CHUNK, CHUNK)
        pltpu.async_copy(ub_hbm.at[uidx_v.at[s]], ub_v.at[s], sem)
        pltpu.async_copy(ib_hbm.at[iidx_v.at[s]], ib_v.at[s], sem)
    pltpu.make_async_copy(ub_hbm.at[pl.ds(0, BPW)], ub_v, sem).wait()
    pltpu.make_async_copy(ib_hbm.at[pl.ds(0, BPW)], ib_v, sem).wait()

    for c in range(NCH):
        idx_u = uidx_v.at[pl.ds(c * CHUNK, CHUNK)]
        idx_i = iidx_v.at[pl.ds(c * CHUNK, CHUNK)]
        cps = [
            pltpu.async_copy(uf_hbm.at[idx_u], urows_v, sem),
            pltpu.async_copy(if_hbm.at[idx_i], irows_v, sem),
        ]
        for cp in cps:
            cp.wait()

        for g in range(CHUNK // L):
            rows = lax.iota(jnp.int32, L) + g * L
            acc0 = ub_v[pl.ds(c * CHUNK + g * L, L)] + ib_v[
                pl.ds(c * CHUNK + g * L, L)]

            def body(d, acc, rows=rows):
                dd = jnp.full((L,), 0, jnp.int32) + d
                uv = plsc.load_gather(urows_v, [rows, dd])
                iv = plsc.load_gather(irows_v, [rows, dd])
                return acc + uv * iv

            out_v[pl.ds(c * CHUNK + g * L, L)] = lax.fori_loop(
                0, D, body, acc0)

    pltpu.sync_copy(out_v, out_hbm.at[pl.ds(base, BPW)])


def kernel(user, item, user_factors, item_factors, user_biases, item_biases):
    uf_rm = _transpose_tc(user_factors.T)
    if_rm = _transpose_tc(item_factors.T)
    return _mf_kernel(user.astype(jnp.int32), item.astype(jnp.int32),
                      uf_rm, if_rm,
                      user_biases.reshape(-1), item_biases.reshape(-1))


# speedup vs baseline: 2.0982x; 2.0982x over previous
import functools

import jax
import jax.numpy as jnp
from jax import lax
from jax.experimental import pallas as pl
from jax.experimental.pallas import tpu as pltpu
from jax.experimental.pallas import tpu_sc as plsc

B = 16384
D = 64
V = 1000000
NC = 2
NS = 16
NW = NC * NS
BPW = B // NW
CHUNK = 128
NCH = BPW // CHUNK
L = 16
TBLK = 8192


def _transpose_body(inT_ref, out_ref):
    eye = (lax.broadcasted_iota(jnp.int32, (D, D), 0)
           == lax.broadcasted_iota(jnp.int32, (D, D), 1)).astype(jnp.float32)
    out_ref[...] = lax.dot_general(
        inT_ref[...], eye, (((0,), (0,)), ((), ())),
        precision=lax.Precision.HIGHEST,
        preferred_element_type=jnp.float32)


_transpose_tc = pl.pallas_call(
    _transpose_body,
    out_shape=jax.ShapeDtypeStruct((V, D), jnp.float32),
    grid=(pl.cdiv(V, TBLK),),
    in_specs=[pl.BlockSpec((D, TBLK), lambda i: (0, i))],
    out_specs=pl.BlockSpec((TBLK, D), lambda i: (i, 0)),
)

_mesh = plsc.VectorSubcoreMesh(core_axis_name="c", subcore_axis_name="s")


@functools.partial(
    pl.kernel,
    out_type=jax.ShapeDtypeStruct((B,), jnp.float32),
    mesh=_mesh,
    compiler_params=pltpu.CompilerParams(
        needs_layout_passes=False, use_tc_tiling_on_sc=False),
    scratch_types=[
        pltpu.VMEM((BPW,), jnp.int32),
        pltpu.VMEM((BPW,), jnp.int32),
        pltpu.VMEM((CHUNK, D), jnp.float32),
        pltpu.VMEM((CHUNK, D), jnp.float32),
        pltpu.VMEM((BPW,), jnp.float32),
        pltpu.VMEM((BPW,), jnp.float32),
        pltpu.VMEM((BPW,), jnp.float32),
        pltpu.SemaphoreType.DMA,
    ],
)
def _mf_kernel(user_hbm, item_hbm, uf_hbm, if_hbm, ub_hbm, ib_hbm, out_hbm,
               uidx_v, iidx_v, urows_v, irows_v, ub_v, ib_v, out_v, sem):
    wid = lax.axis_index("s") * NC + lax.axis_index("c")
    base = wid * BPW

    pltpu.sync_copy(user_hbm.at[pl.ds(base, BPW)], uidx_v)
    pltpu.sync_copy(item_hbm.at[pl.ds(base, BPW)], iidx_v)

    for c in range(NCH):
        s = pl.ds(c * CHUNK, CHUNK)
        pltpu.async_copy(ub_hbm.at[uidx_v.at[s]], ub_v.at[s], sem)
        pltpu.async_copy(ib_hbm.at[iidx_v.at[s]], ib_v.at[s], sem)
    pltpu.make_async_copy(ub_hbm.at[pl.ds(0, BPW)], ub_v, sem).wait()
    pltpu.make_async_copy(ib_hbm.at[pl.ds(0, BPW)], ib_v, sem).wait()

    for c in range(NCH):
        idx_u = uidx_v.at[pl.ds(c * CHUNK, CHUNK)]
        idx_i = iidx_v.at[pl.ds(c * CHUNK, CHUNK)]
        cps = [
            pltpu.async_copy(uf_hbm.at[idx_u], urows_v, sem),
            pltpu.async_copy(if_hbm.at[idx_i], irows_v, sem),
        ]
        for cp in cps:
            cp.wait()

        for g in range(CHUNK // L):
            rows = lax.iota(jnp.int32, L) + g * L
            acc0 = ub_v[pl.ds(c * CHUNK + g * L, L)] + ib_v[
                pl.ds(c * CHUNK + g * L, L)]

            def body(d, acc, rows=rows):
                dd = jnp.full((L,), 0, jnp.int32) + d
                uv = plsc.load_gather(urows_v, [rows, dd])
                iv = plsc.load_gather(irows_v, [rows, dd])
                return acc + uv * iv

            out_v[pl.ds(c * CHUNK + g * L, L)] = lax.fori_loop(
                0, D, body, acc0)

    pltpu.sync_copy(out_v, out_hbm.at[pl.ds(base, BPW)])


def kernel(user, item, user_factors, item_factors, user_biases, item_biases):
    uf_rm = _transpose_tc(user_factors.T)
    if_rm = _transpose_tc(item_factors.T)
    return _mf_kernel(user.astype(jnp.int32), item.astype(jnp.int32),
                      uf_rm, if_rm,
                      user_biases.reshape(-1), item_biases.reshape(-1))


# speedup vs baseline: 2.3165x; 1.1040x over previous
import functools

import jax
import jax.numpy as jnp
from jax import lax
from jax.experimental import pallas as pl
from jax.experimental.pallas import tpu as pltpu
from jax.experimental.pallas import tpu_sc as plsc

B = 16384
D = 64
V = 1000000
NC = 2
NS = 16
NW = NC * NS
BPW = B // NW
CHUNK = 128
NCH = BPW // CHUNK
L = 16
TBLK = 8192


def _transpose_body(inT_ref, out_ref):
    x = inT_ref[...].reshape(D, TBLK // 128, 128)
    out_ref[...] = x.transpose(1, 2, 0).reshape(TBLK, D)


_transpose_tc = pl.pallas_call(
    _transpose_body,
    out_shape=jax.ShapeDtypeStruct((V, D), jnp.float32),
    grid=(pl.cdiv(V, TBLK),),
    in_specs=[pl.BlockSpec((D, TBLK), lambda i: (0, i))],
    out_specs=pl.BlockSpec((TBLK, D), lambda i: (i, 0)),
)

_mesh = plsc.VectorSubcoreMesh(core_axis_name="c", subcore_axis_name="s")


@functools.partial(
    pl.kernel,
    out_type=jax.ShapeDtypeStruct((B,), jnp.float32),
    mesh=_mesh,
    compiler_params=pltpu.CompilerParams(
        needs_layout_passes=False, use_tc_tiling_on_sc=False),
    scratch_types=[
        pltpu.VMEM((BPW,), jnp.int32),
        pltpu.VMEM((BPW,), jnp.int32),
        pltpu.VMEM((CHUNK, D), jnp.float32),
        pltpu.VMEM((CHUNK, D), jnp.float32),
        pltpu.VMEM((BPW,), jnp.float32),
        pltpu.VMEM((BPW,), jnp.float32),
        pltpu.VMEM((BPW,), jnp.float32),
        pltpu.SemaphoreType.DMA,
    ],
)
def _mf_kernel(user_hbm, item_hbm, uf_hbm, if_hbm, ub_hbm, ib_hbm, out_hbm,
               uidx_v, iidx_v, urows_v, irows_v, ub_v, ib_v, out_v, sem):
    wid = lax.axis_index("s") * NC + lax.axis_index("c")
    base = wid * BPW

    pltpu.sync_copy(user_hbm.at[pl.ds(base, BPW)], uidx_v)
    pltpu.sync_copy(item_hbm.at[pl.ds(base, BPW)], iidx_v)

    for c in range(NCH):
        s = pl.ds(c * CHUNK, CHUNK)
        pltpu.async_copy(ub_hbm.at[uidx_v.at[s]], ub_v.at[s], sem)
        pltpu.async_copy(ib_hbm.at[iidx_v.at[s]], ib_v.at[s], sem)
    pltpu.make_async_copy(ub_hbm.at[pl.ds(0, BPW)], ub_v, sem).wait()
    pltpu.make_async_copy(ib_hbm.at[pl.ds(0, BPW)], ib_v, sem).wait()

    for c in range(NCH):
        idx_u = uidx_v.at[pl.ds(c * CHUNK, CHUNK)]
        idx_i = iidx_v.at[pl.ds(c * CHUNK, CHUNK)]
        cps = [
            pltpu.async_copy(uf_hbm.at[idx_u], urows_v, sem),
            pltpu.async_copy(if_hbm.at[idx_i], irows_v, sem),
        ]
        for cp in cps:
            cp.wait()

        for g in range(CHUNK // L):
            rows = lax.iota(jnp.int32, L) + g * L
            acc0 = ub_v[pl.ds(c * CHUNK + g * L, L)] + ib_v[
                pl.ds(c * CHUNK + g * L, L)]

            def body(d, acc, rows=rows):
                dd = jnp.full((L,), 0, jnp.int32) + d
                uv = plsc.load_gather(urows_v, [rows, dd])
                iv = plsc.load_gather(irows_v, [rows, dd])
                return acc + uv * iv

            out_v[pl.ds(c * CHUNK + g * L, L)] = lax.fori_loop(
                0, D, body, acc0)

    pltpu.sync_copy(out_v, out_hbm.at[pl.ds(base, BPW)])


def kernel(user, item, user_factors, item_factors, user_biases, item_biases):
    uf_rm = _transpose_tc(user_factors.T)
    if_rm = _transpose_tc(item_factors.T)
    return _mf_kernel(user.astype(jnp.int32), item.astype(jnp.int32),
                      uf_rm, if_rm,
                      user_biases.reshape(-1), item_biases.reshape(-1))
